# baseline (speedup 1.0000x reference)
import functools

import jax
import jax.numpy as jnp
from jax import lax
from jax.experimental import pallas as pl
from jax.experimental.pallas import tpu as pltpu
from jax.experimental.pallas import tpu_sc as plsc

_LANES = 16
_CHUNK_ROWS = 8
_NBUF = 4
_PREFETCH = 2
_UNROLL = 8
_SC_BATCHES = 1
_TC_BS = 512


def _make_sc_kernel(batch, b0, sc_batches, seq_len, d_model):
    info = plsc.get_sparse_core_info()
    nc, ns = info.num_cores, info.num_subcores
    nw = nc * ns
    total_rows = sc_batches * seq_len
    rows_per_w = total_rows // nw
    w_per_batch = seq_len // rows_per_w
    c = _CHUNK_ROWS
    vecs_per_row = d_model // _LANES
    nchunks = rows_per_w // c
    mesh = plsc.VectorSubcoreMesh(core_axis_name="c", subcore_axis_name="s")

    buf = lambda: pltpu.VMEM((c, d_model), jnp.float32)
    sem = lambda: pltpu.SemaphoreType.DMA

    @functools.partial(
        pl.kernel,
        mesh=mesh,
        out_type=jax.ShapeDtypeStruct((sc_batches, seq_len, d_model), jnp.float32),
        scratch_types=(
            [buf() for _ in range(_NBUF)]
            + [buf() for _ in range(_NBUF)]
            + [sem() for _ in range(3 * _NBUF)]
        ),
    )
    def sc_add(x_hbm, pos_hbm, out_hbm, *scratch):
        xo_bufs = scratch[0:_NBUF]
        p_bufs = scratch[_NBUF:2 * _NBUF]
        sems = scratch[2 * _NBUF:]
        sx = sems[0:_NBUF]
        sp = sems[_NBUF:2 * _NBUF]
        so = sems[2 * _NBUF:]

        wid = lax.axis_index("s") * nc + lax.axis_index("c")
        rb = wid // w_per_batch
        s0 = (wid % w_per_batch) * rows_per_w

        def x_cp(j, u):
            s = s0 + j * c
            return pltpu.make_async_copy(
                x_hbm.at[b0 + rb, pl.ds(s, c)], xo_bufs[u], sx[u])

        def p_cp(j, u):
            s = s0 + j * c
            return pltpu.make_async_copy(pos_hbm.at[pl.ds(s, c)], p_bufs[u], sp[u])

        def o_cp(j, u):
            s = s0 + j * c
            return pltpu.make_async_copy(
                xo_bufs[u], out_hbm.at[rb, pl.ds(s, c)], so[u])

        for u in range(_PREFETCH):
            x_cp(u, u).start()
            p_cp(u, u).start()

        def round_body(t, carry):
            for u in range(_NBUF):
                j = t * _NBUF + u
                x_cp(j, u).wait()
                p_cp(j, u).wait()

                xo_v, p_v = xo_bufs[u], p_bufs[u]

                @plsc.parallel_loop(0, c * vecs_per_row, step=1, unroll=_UNROLL)
                def add_body(i):
                    r = i // vecs_per_row
                    k = lax.rem(i, vecs_per_row) * _LANES
                    sl = pl.ds(k, _LANES)
                    plsc.addupdate(xo_v.at[r, sl], p_v[r, sl])

                o_cp(j, u).start()

                jn = j + _PREFETCH
                un = (u + _PREFETCH) % _NBUF

                @pl.when(jn < nchunks)
                def _():
                    @pl.when(jn >= _NBUF)
                    def _():
                        o_cp(jn - _NBUF, un).wait()

                    x_cp(jn, un).start()
                    p_cp(jn, un).start()
            return carry

        lax.fori_loop(0, nchunks // _NBUF, round_body, 0)

        for u in range(_NBUF):
            j = nchunks - _NBUF + u
            o_cp(j, u).wait()

    return sc_add


def _tc_add_body(x_ref, pos_ref, out_ref):
    out_ref[...] = x_ref[...] + pos_ref[...]


def _tc_add(x, pos, tc_batches):
    batch, seq_len, d_model = x.shape
    bs = _TC_BS if seq_len % _TC_BS == 0 else seq_len
    return pl.pallas_call(
        _tc_add_body,
        grid=(seq_len // bs,),
        in_specs=[
            pl.BlockSpec((tc_batches, bs, d_model), lambda s: (0, s, 0)),
            pl.BlockSpec((bs, d_model), lambda s: (s, 0)),
        ],
        out_specs=pl.BlockSpec((tc_batches, bs, d_model), lambda s: (0, s, 0)),
        out_shape=jax.ShapeDtypeStruct((tc_batches, seq_len, d_model), x.dtype),
    )(x, pos)


def kernel(x, pos_weight):
    batch, seq_len, d_model = x.shape
    pos = pos_weight[:seq_len]
    tc_batches = batch - _SC_BATCHES
    sc = _make_sc_kernel(batch, tc_batches, _SC_BATCHES, seq_len, d_model)
    sc_out = sc(x, pos)
    tc_out = _tc_add(x, pos, tc_batches)
    return jnp.concatenate([tc_out, sc_out], axis=0)


# speedup vs baseline: 1.3209x; 1.3209x over previous
import functools

import jax
import jax.numpy as jnp
from jax import lax
from jax.experimental import pallas as pl
from jax.experimental.pallas import tpu as pltpu
from jax.experimental.pallas import tpu_sc as plsc

_LANES = 16
_CHUNK_ROWS = 16
_UNROLL = 8


def _make_sc_kernel(batch, seq_len, d_model):
    info = plsc.get_sparse_core_info()
    nc, ns = info.num_cores, info.num_subcores
    nw = nc * ns
    rows_per_w = seq_len // nw
    c = _CHUNK_ROWS
    vecs_per_row = d_model // _LANES
    nchunks = rows_per_w // c
    nslots = nchunks * batch
    nbuf = batch
    mesh = plsc.VectorSubcoreMesh(core_axis_name="c", subcore_axis_name="s")

    buf = lambda: pltpu.VMEM((c, d_model), jnp.float32)
    sem = lambda: pltpu.SemaphoreType.DMA

    @functools.partial(
        pl.kernel,
        mesh=mesh,
        out_type=jax.ShapeDtypeStruct((batch, seq_len, d_model), jnp.float32),
        scratch_types=(
            [buf() for _ in range(nbuf)]
            + [buf(), buf()]
            + [sem() for _ in range(2 * nbuf + 2)]
        ),
    )
    def sc_add(x_hbm, pos_hbm, out_hbm, *scratch):
        xo_bufs = scratch[0:nbuf]
        p_bufs = scratch[nbuf:nbuf + 2]
        sems = scratch[nbuf + 2:]
        sx = sems[0:nbuf]
        so = sems[nbuf:2 * nbuf]
        sp = sems[2 * nbuf:]

        wid = lax.axis_index("s") * nc + lax.axis_index("c")
        s0 = wid * rows_per_w

        def x_cp(k):
            j, b, u = k // batch, k % batch, k % nbuf
            return pltpu.make_async_copy(
                x_hbm.at[b, pl.ds(s0 + j * c, c)], xo_bufs[u], sx[u])

        def o_cp(k):
            j, b, u = k // batch, k % batch, k % nbuf
            return pltpu.make_async_copy(
                xo_bufs[u], out_hbm.at[b, pl.ds(s0 + j * c, c)], so[u])

        def p_cp(j):
            return pltpu.make_async_copy(
                pos_hbm.at[pl.ds(s0 + j * c, c)], p_bufs[j % 2], sp[j % 2])

        p_cp(0).start()
        x_cp(0).start()
        x_cp(1).start()

        for k in range(nslots):
            j, b = k // batch, k % batch
            if b == 0:
                p_cp(j).wait()
                if j + 1 < nchunks:
                    p_cp(j + 1).start()
            x_cp(k).wait()

            xo_v, p_v = xo_bufs[k % nbuf], p_bufs[j % 2]

            @plsc.parallel_loop(0, c * vecs_per_row, step=1, unroll=_UNROLL)
            def add_body(i):
                r = i // vecs_per_row
                col = lax.rem(i, vecs_per_row) * _LANES
                sl = pl.ds(col, _LANES)
                plsc.addupdate(xo_v.at[r, sl], p_v[r, sl])

            o_cp(k).start()

            if k + 2 < nslots:
                if k - 2 >= 0:
                    o_cp(k - 2).wait()
                x_cp(k + 2).start()

        o_cp(nslots - 2).wait()
        o_cp(nslots - 1).wait()

    return sc_add


def kernel(x, pos_weight):
    batch, seq_len, d_model = x.shape
    sc = _make_sc_kernel(batch, seq_len, d_model)
    return sc(x, pos_weight[:seq_len])


# speedup vs baseline: 1.3732x; 1.0396x over previous
import functools

import jax
import jax.numpy as jnp
from jax import lax
from jax.experimental import pallas as pl
from jax.experimental.pallas import tpu as pltpu
from jax.experimental.pallas import tpu_sc as plsc

_LANES = 16
_CHUNK_ROWS = 16
_NBUF = 5
_PREFETCH = 3
_UNROLL = 8


def _make_sc_kernel(batch, seq_len, d_model):
    info = plsc.get_sparse_core_info()
    nc, ns = info.num_cores, info.num_subcores
    nw = nc * ns
    rows_per_w = seq_len // nw
    c = _CHUNK_ROWS
    vecs_per_row = d_model // _LANES
    nchunks = rows_per_w // c
    nslots = nchunks * batch
    nbuf = _NBUF
    mesh = plsc.VectorSubcoreMesh(core_axis_name="c", subcore_axis_name="s")

    buf = lambda: pltpu.VMEM((c, d_model), jnp.float32)
    sem = lambda: pltpu.SemaphoreType.DMA

    @functools.partial(
        pl.kernel,
        mesh=mesh,
        out_type=jax.ShapeDtypeStruct((batch, seq_len, d_model), jnp.float32),
        scratch_types=(
            [buf() for _ in range(nbuf)]
            + [buf(), buf()]
            + [sem() for _ in range(2 * nbuf + 2)]
        ),
    )
    def sc_add(x_hbm, pos_hbm, out_hbm, *scratch):
        xo_bufs = scratch[0:nbuf]
        p_bufs = scratch[nbuf:nbuf + 2]
        sems = scratch[nbuf + 2:]
        sx = sems[0:nbuf]
        so = sems[nbuf:2 * nbuf]
        sp = sems[2 * nbuf:]

        wid = lax.axis_index("s") * nc + lax.axis_index("c")
        s0 = wid * rows_per_w

        def x_cp(k):
            j, b, u = k // batch, k % batch, k % nbuf
            return pltpu.make_async_copy(
                x_hbm.at[b, pl.ds(s0 + j * c, c)], xo_bufs[u], sx[u])

        def o_cp(k):
            j, b, u = k // batch, k % batch, k % nbuf
            return pltpu.make_async_copy(
                xo_bufs[u], out_hbm.at[b, pl.ds(s0 + j * c, c)], so[u])

        def p_cp(j):
            return pltpu.make_async_copy(
                pos_hbm.at[pl.ds(s0 + j * c, c)], p_bufs[j % 2], sp[j % 2])

        p_cp(0).start()
        for k0 in range(_PREFETCH):
            x_cp(k0).start()

        for k in range(nslots):
            j, b = k // batch, k % batch
            if b == 0:
                p_cp(j).wait()
                if j + 1 < nchunks:
                    p_cp(j + 1).start()
            x_cp(k).wait()

            xo_v, p_v = xo_bufs[k % nbuf], p_bufs[j % 2]

            @plsc.parallel_loop(0, c * vecs_per_row, step=1, unroll=_UNROLL)
            def add_body(i):
                r = i // vecs_per_row
                col = lax.rem(i, vecs_per_row) * _LANES
                sl = pl.ds(col, _LANES)
                plsc.addupdate(xo_v.at[r, sl], p_v[r, sl])

            o_cp(k).start()

            if k + _PREFETCH < nslots:
                if k + _PREFETCH - nbuf >= 0:
                    o_cp(k + _PREFETCH - nbuf).wait()
                x_cp(k + _PREFETCH).start()

        for m in range(max(0, nslots - nbuf), nslots):
            o_cp(m).wait()

    return sc_add


def kernel(x, pos_weight):
    batch, seq_len, d_model = x.shape
    sc = _make_sc_kernel(batch, seq_len, d_model)
    return sc(x, pos_weight[:seq_len])
